# baseline (speedup 1.0000x reference)
import functools

import jax
import jax.numpy as jnp
from jax import lax
from jax.experimental import pallas as pl
from jax.experimental.pallas import tpu as pltpu
from jax.experimental.pallas import tpu_sc as plsc

_VOCAB = 1000000
_EMB = 64
_B = 4096
_L = 200
_SCALE = 8.0

_NC = 2
_NS = 16
_NW = _NC * _NS

_N = _B * _L
_PER_W = _N // _NW

_NBLK = (_VOCAB + 127) // 128
_SCR_ROWS = ((_NBLK * 128) // 2)
_BLK_LO = _NBLK // _NW
_BLK_EXTRA = _NBLK - _BLK_LO * _NW

_C = 256
_NCHUNK = _PER_W // _C


def _transpose_table(table_t):
    mesh = plsc.VectorSubcoreMesh(core_axis_name="c", subcore_axis_name="s")

    @functools.partial(
        pl.kernel,
        mesh=mesh,
        compiler_params=pltpu.CompilerParams(needs_layout_passes=False),
        out_type=jax.ShapeDtypeStruct((_SCR_ROWS, 128), jnp.float32),
        scratch_types=[
            pltpu.VMEM((_EMB, 128), jnp.float32),
            pltpu.VMEM((_EMB, 128), jnp.float32),
        ],
    )
    def ka(tab_hbm, scr_hbm, vin, vout):
        wid = lax.axis_index("s") * _NC + lax.axis_index("c")
        nblk = jnp.where(wid < _BLK_EXTRA, _BLK_LO + 1, _BLK_LO)
        blk0 = wid * _BLK_LO + jnp.minimum(wid, _BLK_EXTRA)

        lane = lax.iota(jnp.int32, 16)
        parity64 = (lane & 1) * 64
        half_lane = lane >> 1

        def blk_body(i, carry):
            blk = blk0 + i
            v0 = pl.multiple_of(blk * 128, 128)
            pltpu.sync_copy(tab_hbm.at[:, pl.ds(v0, 128)], vin)

            def col_body(e, c):
                cvec = parity64 + e
                for g in range(8):
                    x = vin[e, pl.ds(16 * g, 16)] * _SCALE
                    pvec = half_lane + 8 * g
                    plsc.store_scatter(vout, [pvec, cvec], x)
                return c

            lax.fori_loop(0, _EMB, col_body, 0)
            p0 = pl.multiple_of(v0 // 2, 64)
            pltpu.sync_copy(vout, scr_hbm.at[pl.ds(p0, 64)])
            return carry

        lax.fori_loop(0, nblk, blk_body, 0)

    return ka(table_t)


def _gather_rows(tokens_flat, scr):
    mesh = plsc.VectorSubcoreMesh(core_axis_name="c", subcore_axis_name="s")

    @functools.partial(
        pl.kernel,
        mesh=mesh,
        compiler_params=pltpu.CompilerParams(needs_layout_passes=False),
        out_type=jax.ShapeDtypeStruct((_N, _EMB), jnp.float32),
        scratch_types=[
            pltpu.VMEM((_C,), jnp.int32),
            pltpu.VMEM((_C,), jnp.int32),
            pltpu.VMEM((_C, 128), jnp.float32),
            pltpu.VMEM((_C, _EMB), jnp.float32),
            pltpu.SemaphoreType.DMA,
        ],
    )
    def kb(tok_hbm, scr_hbm, out_hbm, idx_v, idxp_v, rows_v, out_v, sem):
        wid = lax.axis_index("s") * _NC + lax.axis_index("c")
        base = wid * _PER_W
        lane = lax.iota(jnp.int32, 16)

        def chunk16(g, carry):
            off = pl.multiple_of(base + g * _C, _C)
            pltpu.sync_copy(tok_hbm.at[pl.ds(off, _C)], idx_v)
            for j in range(_C // 16):
                idxp_v[pl.ds(16 * j, 16)] = idx_v[pl.ds(16 * j, 16)] >> 1
            pltpu.async_copy(scr_hbm.at[idxp_v], rows_v, sem).wait()

            def group_body(gi, c):
                r0 = gi * 16
                rvec = r0 + lane
                hvec = (plsc.load_gather(idx_v, [rvec]) & 1) * 64
                for col in range(_EMB):
                    x = plsc.load_gather(rows_v, [rvec, hvec + col])
                    plsc.store_scatter(
                        out_v, [rvec, jnp.full((16,), col, jnp.int32)], x
                    )
                return c

            lax.fori_loop(0, _C // 16, group_body, 0)
            pltpu.sync_copy(out_v, out_hbm.at[pl.ds(off, _C)])
            return carry

        lax.fori_loop(0, _NCHUNK, chunk16, 0)

    return kb(tokens_flat, scr)


def kernel(tokens, table):
    tok = tokens.reshape(-1).astype(jnp.int32)
    table_t = table.T
    scr = _transpose_table(table_t)
    out = _gather_rows(tok, scr)
    return out.reshape(_B, _L, _EMB)


# speedup vs baseline: 3.0136x; 3.0136x over previous
import functools

import jax
import jax.numpy as jnp
from jax import lax
from jax.experimental import pallas as pl
from jax.experimental.pallas import tpu as pltpu
from jax.experimental.pallas import tpu_sc as plsc

_VOCAB = 1000000
_EMB = 64
_B = 4096
_L = 200
_SCALE = 8.0

_NC = 2
_NS = 16
_NW = _NC * _NS

_NBLK = (_VOCAB + 127) // 128
_SCR_ROWS = (_NBLK * 128) // 2
_BLK_LO = _NBLK // _NW
_BLK_EXTRA = _NBLK - _BLK_LO * _NW


def _transpose_table(table_t):
    mesh = plsc.VectorSubcoreMesh(core_axis_name="c", subcore_axis_name="s")

    @functools.partial(
        pl.kernel,
        mesh=mesh,
        compiler_params=pltpu.CompilerParams(needs_layout_passes=False),
        out_type=jax.ShapeDtypeStruct((_SCR_ROWS, 128), jnp.float32),
        scratch_types=[
            pltpu.VMEM((2, _EMB, 128), jnp.float32),
            pltpu.VMEM((2, _EMB, 128), jnp.float32),
            pltpu.SemaphoreType.DMA,
            pltpu.SemaphoreType.DMA,
        ],
    )
    def ka(tab_hbm, scr_hbm, vin, vout, semi0, semi1):
        wid = lax.axis_index("s") * _NC + lax.axis_index("c")
        nblk = jnp.where(wid < _BLK_EXTRA, _BLK_LO + 1, _BLK_LO)
        blk0 = wid * _BLK_LO + jnp.minimum(wid, _BLK_EXTRA)

        lane = lax.iota(jnp.int32, 16)
        vvecs = [16 * g + lane for g in range(8)]
        vvecs64 = [(16 * g + lane) * _EMB for g in range(8)]

        def vsrc(i):
            v0 = pl.multiple_of((blk0 + i) * 128, 128)
            return tab_hbm.at[:, pl.ds(v0, 128)]

        def fire(i, buf, sem):
            return pltpu.async_copy(vsrc(i), vin.at[buf], sem)

        def transpose_block(i, buf):
            for g in range(8):
                vvec = vvecs[g]
                vvec64 = vvecs64[g]

                def e_body(e0, carry):
                    evec = (e0 + lane) & (_EMB - 1)
                    x = plsc.load_gather(vin.at[buf], [evec, vvec]) * _SCALE
                    flat = vvec64 + evec
                    plsc.store_scatter(
                        vout.at[buf], [flat >> 7, flat & 127], x
                    )
                    return carry

                lax.fori_loop(0, _EMB, e_body, 0)
            p0 = pl.multiple_of((blk0 + i) * 64, 64)
            pltpu.sync_copy(vout.at[buf], scr_hbm.at[pl.ds(p0, 64)])

        fire(0, 0, semi0)

        def pair(j, carry):
            i0 = 2 * j

            @pl.when(i0 + 1 < nblk)
            def _():
                fire(i0 + 1, 1, semi1)

            @pl.when(i0 < nblk)
            def _():
                pltpu.make_async_copy(vsrc(i0), vin.at[0], semi0).wait()
                transpose_block(i0, 0)

            @pl.when(i0 + 2 < nblk)
            def _():
                fire(i0 + 2, 0, semi0)

            @pl.when(i0 + 1 < nblk)
            def _():
                pltpu.make_async_copy(vsrc(i0 + 1), vin.at[1], semi1).wait()
                transpose_block(i0 + 1, 1)

            return carry

        lax.fori_loop(0, (_BLK_LO + 2) // 2, pair, 0)

    return ka(table_t)


def _gather_rows(tokens_t, scr):
    mesh = plsc.VectorSubcoreMesh(core_axis_name="c", subcore_axis_name="s")

    @functools.partial(
        pl.kernel,
        mesh=mesh,
        compiler_params=pltpu.CompilerParams(needs_layout_passes=False),
        out_type=jax.ShapeDtypeStruct((_L, 8, _NW, 8, 128), jnp.float32),
        scratch_types=[
            pltpu.VMEM((2, 128), jnp.int32),
            pltpu.VMEM((2, 128), jnp.int32),
            pltpu.VMEM((2, 128), jnp.int32),
            pltpu.VMEM((2, 128, 128), jnp.float32),
            pltpu.VMEM((2, 8, 8, 128), jnp.float32),
            pltpu.SemaphoreType.DMA,
            pltpu.SemaphoreType.DMA,
            pltpu.SemaphoreType.DMA,
            pltpu.SemaphoreType.DMA,
        ],
    )
    def kb(tok_hbm, scr_hbm, out_hbm, idx_v, idxp_v, hv_v, rows_v, ob_v,
           semg0, semg1, semw0, semw1):
        wid = lax.axis_index("s") * _NC + lax.axis_index("c")
        b0 = pl.multiple_of(wid * 128, 128)
        lane = lax.iota(jnp.int32, 16)
        bvecs = [16 * g + lane for g in range(8)]

        def fire(l, buf, sem):
            pltpu.sync_copy(tok_hbm.at[l, pl.ds(b0, 128)], idx_v.at[buf])
            for g in range(8):
                sl = pl.ds(16 * g, 16)
                idxp_v[buf, sl] = idx_v[buf, sl] >> 1
            return pltpu.async_copy(
                scr_hbm.at[idxp_v.at[buf]], rows_v.at[buf], sem
            )

        def wait_gather(buf, sem):
            pltpu.make_async_copy(
                scr_hbm.at[idxp_v.at[buf]], rows_v.at[buf], sem
            ).wait()

        def owin(l):
            return out_hbm.at[l, :, wid, :, :]

        def extract(l, buf, semw):
            for g in range(8):
                sl = pl.ds(16 * g, 16)
                hv_v[buf, sl] = (idx_v[buf, sl] & 1) * _EMB
            ob2d = ob_v.at[buf].reshape(_EMB, 128)
            for g in range(8):
                bvec = bvecs[g]
                hvec = plsc.load_gather(hv_v.at[buf], [bvec])

                def e_body(e0, carry):
                    evec = (e0 + lane) & (_EMB - 1)
                    x = plsc.load_gather(rows_v.at[buf], [bvec, hvec + evec])
                    plsc.store_scatter(ob2d, [evec, bvec], x)
                    return carry

                lax.fori_loop(0, _EMB, e_body, 0)
            return pltpu.async_copy(ob_v.at[buf], owin(l), semw)

        def drain_write(l, buf, semw):
            pltpu.make_async_copy(ob_v.at[buf], owin(l), semw).wait()

        fire(0, 0, semg0)

        def pair(j, carry):
            l0 = 2 * j
            fire(l0 + 1, 1, semg1)
            wait_gather(0, semg0)

            @pl.when(l0 >= 2)
            def _():
                drain_write(l0 - 2, 0, semw0)

            extract(l0, 0, semw0)

            @pl.when(l0 + 2 < _L)
            def _():
                fire(l0 + 2, 0, semg0)

            wait_gather(1, semg1)

            @pl.when(l0 >= 2)
            def _():
                drain_write(l0 - 1, 1, semw1)

            extract(l0 + 1, 1, semw1)
            return carry

        lax.fori_loop(0, _L // 2, pair, 0)
        drain_write(_L - 2, 0, semw0)
        drain_write(_L - 1, 1, semw1)

    return kb(tokens_t, scr)


def kernel(tokens, table):
    table_t = table.T
    tokens_t = tokens.T.astype(jnp.int32)
    scr = _transpose_table(table_t)
    out5 = _gather_rows(tokens_t, scr)
    return out5.transpose(2, 4, 0, 1, 3).reshape(_B, _L, _EMB)
